# baseline (speedup 1.0000x reference)
import functools

import jax
import jax.numpy as jnp
from jax import lax
from jax.experimental import pallas as pl
from jax.experimental.pallas import tpu as pltpu
from jax.experimental.pallas import tpu_sc as plsc

N_NODES_K = 50000
HID = 64
N_PAD = 50176
ROWS = N_PAD // 128
N_EDGES_K = 800000
NC, NS = 2, 16
NW = NC * NS
CHUNK = 1792
NCHUNK = 14
E_TILE = NCHUNK * CHUNK
E_PAD = NW * E_TILE
VSTEPS = CHUNK // 16
SROWS = CHUNK // 128
SLICE = N_PAD // NS


NBUF = 2
UNROLL = 8


def _seg_body(x_hbm, src_hbm, w_hbm, dst_hbm, out_hbm,
              x_v, acc_v, x_sem, in_sems, *bufs):
    src_v = bufs[0:NBUF]
    w_v = bufs[NBUF:2 * NBUF]
    dst_v = bufs[2 * NBUF:3 * NBUF]
    cid = lax.axis_index("c")
    sid = lax.axis_index("s")
    wid = cid * NS + sid

    x_cp = pltpu.async_copy(x_hbm, x_v, x_sem)

    def zbody(i, _):
        for u in range(UNROLL):
            acc_v[pl.ds(i * (16 * UNROLL) + u * 16, 16)] = (
                jnp.zeros((16,), jnp.float32))
        return 0
    lax.fori_loop(0, N_PAD // (16 * UNROLL), zbody, 0)
    x_cp.wait()

    def fire_inputs(c):
        b = c % NBUF
        base = wid * E_TILE + c * CHUNK
        return [
            pltpu.async_copy(src_hbm.at[pl.ds(base, CHUNK)],
                             src_v[b], in_sems.at[b]),
            pltpu.async_copy(w_hbm.at[pl.ds(base, CHUNK)],
                             w_v[b], in_sems.at[b]),
            pltpu.async_copy(dst_hbm.at[pl.ds(base, CHUNK)],
                             dst_v[b], in_sems.at[b]),
        ]

    in_cps = {0: fire_inputs(0), 1: fire_inputs(1)}
    for c in range(NCHUNK):
        b = c % NBUF
        for cp in in_cps.pop(c):
            cp.wait()

        def vbody(j, _):
            for u in range(UNROLL):
                o = j * (16 * UNROLL) + u * 16
                s = src_v[b][pl.ds(o, 16)]
                wv = w_v[b][pl.ds(o, 16)]
                d = dst_v[b][pl.ds(o, 16)]
                xv = plsc.load_gather(x_v, [s])
                plsc.addupdate_scatter(acc_v, [d], xv * wv)
            return 0
        lax.fori_loop(0, VSTEPS // UNROLL, vbody, 0)

        if c + 2 < NCHUNK:
            in_cps[c + 2] = fire_inputs(c + 2)

    pltpu.sync_copy(acc_v, out_hbm.at[pl.ds(wid * N_PAD, N_PAD)])


_seg_kernel = pl.kernel(
    _seg_body,
    out_type=jax.ShapeDtypeStruct((NW * N_PAD,), jnp.float32),
    mesh=plsc.VectorSubcoreMesh(core_axis_name="c", subcore_axis_name="s",
                                num_cores=NC, num_subcores=NS),
    compiler_params=pltpu.CompilerParams(needs_layout_passes=False),
    scratch_types=[
        pltpu.VMEM((N_PAD,), jnp.float32),
        pltpu.VMEM((N_PAD,), jnp.float32),
        pltpu.SemaphoreType.DMA,
        pltpu.SemaphoreType.DMA((NBUF,)),
    ] + [pltpu.VMEM((CHUNK,), jnp.int32)] * NBUF
      + [pltpu.VMEM((CHUNK,), jnp.float32)] * NBUF
      + [pltpu.VMEM((CHUNK,), jnp.int32)] * NBUF,
)


def _sum_partials(part_ref):
    t = part_ref[:ROWS, :]
    for k in range(1, NW):
        t = t + part_ref[k * ROWS:(k + 1) * ROWS, :]
    return t


def _dense_body(part_ref, x_ref, a_ref, b_ref, c_ref, d_ref, e_ref,
                p_ref, r_ref):
    a1 = _sum_partials(part_ref)
    xv = x_ref[...]

    def body(k, carry):
        pacc, racc = carry
        h = jnp.maximum(a1 * a_ref[k] + xv * b_ref[k] + c_ref[k], 0.0)
        return (pacc + d_ref[k] * h, racc + e_ref[k] * h)

    z = jnp.zeros((ROWS, 128), jnp.float32)
    pv, rv = lax.fori_loop(0, HID, body, (z, z))
    p_ref[...] = pv
    r_ref[...] = rv


_dense = pl.pallas_call(
    _dense_body,
    out_shape=(jax.ShapeDtypeStruct((ROWS, 128), jnp.float32),
               jax.ShapeDtypeStruct((ROWS, 128), jnp.float32)),
    in_specs=[pl.BlockSpec(memory_space=pltpu.VMEM),
              pl.BlockSpec(memory_space=pltpu.VMEM)] +
             [pl.BlockSpec(memory_space=pltpu.SMEM)] * 5,
    out_specs=(pl.BlockSpec(memory_space=pltpu.VMEM),
               pl.BlockSpec(memory_space=pltpu.VMEM)),
)


def _final_body(part_ref, r_ref, b2_ref, o_ref):
    a2 = _sum_partials(part_ref)
    o_ref[...] = jax.nn.sigmoid(a2 + r_ref[...] + b2_ref[0])


_final = pl.pallas_call(
    _final_body,
    out_shape=jax.ShapeDtypeStruct((ROWS, 128), jnp.float32),
    in_specs=[pl.BlockSpec(memory_space=pltpu.VMEM),
              pl.BlockSpec(memory_space=pltpu.VMEM),
              pl.BlockSpec(memory_space=pltpu.SMEM)],
    out_specs=pl.BlockSpec(memory_space=pltpu.VMEM),
)


def kernel(x, edge_index, edge_weight, W1_rel, b1_rel, W1_root,
           W2_rel, b2_rel, W2_root):
    xf = x[:, 0]
    x_pad = jnp.pad(xf, (0, N_PAD - N_NODES_K))
    pad_e = E_PAD - N_EDGES_K
    src_p = jnp.pad(edge_index[0].astype(jnp.int32), (0, pad_e))
    dst_p = jnp.pad(edge_index[1].astype(jnp.int32), (0, pad_e))
    w_p = jnp.pad(edge_weight, (0, pad_e))

    part1 = _seg_kernel(x_pad, src_p, w_p, dst_p)
    p, r = _dense(part1.reshape(NW * ROWS, 128),
                  x_pad.reshape(ROWS, 128),
                  W1_rel[:, 0], W1_root[:, 0], b1_rel,
                  W2_rel[0], W2_root[0])
    part2 = _seg_kernel(p.reshape(-1), src_p, w_p, dst_p)
    out = _final(part2.reshape(NW * ROWS, 128), r, b2_rel)
    return out.reshape(N_PAD)[:N_NODES_K].reshape(N_NODES_K, 1)


# speedup vs baseline: 1.0036x; 1.0036x over previous
import functools

import jax
import jax.numpy as jnp
from jax import lax
from jax.experimental import pallas as pl
from jax.experimental.pallas import tpu as pltpu
from jax.experimental.pallas import tpu_sc as plsc

N_NODES_K = 50000
HID = 64
N_PAD = 50176
ROWS = N_PAD // 128
N_EDGES_K = 800000
NC, NS = 2, 16
NW = NC * NS
CHUNK = 1792
NCHUNK = 14
E_TILE = NCHUNK * CHUNK
E_PAD = NW * E_TILE
VSTEPS = CHUNK // 16
SROWS = CHUNK // 128
SLICE = N_PAD // NS


NBUF = 2
UNROLL = 4


def _seg_body(x_hbm, src_hbm, w_hbm, dst_hbm, out_hbm,
              x_v, acc_v, x_sem, in_sems, *bufs):
    src_v = bufs[0:NBUF]
    w_v = bufs[NBUF:2 * NBUF]
    dst_v = bufs[2 * NBUF:3 * NBUF]
    cid = lax.axis_index("c")
    sid = lax.axis_index("s")
    wid = cid * NS + sid

    x_cp = pltpu.async_copy(x_hbm, x_v, x_sem)

    def zbody(i, _):
        for u in range(UNROLL):
            acc_v[pl.ds(i * (16 * UNROLL) + u * 16, 16)] = (
                jnp.zeros((16,), jnp.float32))
        return 0
    lax.fori_loop(0, N_PAD // (16 * UNROLL), zbody, 0)
    x_cp.wait()

    def fire_inputs(c):
        b = c % NBUF
        base = wid * E_TILE + c * CHUNK
        return [
            pltpu.async_copy(src_hbm.at[pl.ds(base, CHUNK)],
                             src_v[b], in_sems.at[b]),
            pltpu.async_copy(w_hbm.at[pl.ds(base, CHUNK)],
                             w_v[b], in_sems.at[b]),
            pltpu.async_copy(dst_hbm.at[pl.ds(base, CHUNK)],
                             dst_v[b], in_sems.at[b]),
        ]

    in_cps = {0: fire_inputs(0), 1: fire_inputs(1)}
    for c in range(NCHUNK):
        b = c % NBUF
        for cp in in_cps.pop(c):
            cp.wait()

        def vbody(j, _):
            for u in range(UNROLL):
                o = j * (16 * UNROLL) + u * 16
                s = src_v[b][pl.ds(o, 16)]
                wv = w_v[b][pl.ds(o, 16)]
                d = dst_v[b][pl.ds(o, 16)]
                xv = plsc.load_gather(x_v, [s])
                plsc.addupdate_scatter(acc_v, [d], xv * wv)
            return 0
        lax.fori_loop(0, VSTEPS // UNROLL, vbody, 0)

        if c + 2 < NCHUNK:
            in_cps[c + 2] = fire_inputs(c + 2)

    pltpu.sync_copy(acc_v, out_hbm.at[pl.ds(wid * N_PAD, N_PAD)])


_seg_kernel = pl.kernel(
    _seg_body,
    out_type=jax.ShapeDtypeStruct((NW * N_PAD,), jnp.float32),
    mesh=plsc.VectorSubcoreMesh(core_axis_name="c", subcore_axis_name="s",
                                num_cores=NC, num_subcores=NS),
    compiler_params=pltpu.CompilerParams(needs_layout_passes=False),
    scratch_types=[
        pltpu.VMEM((N_PAD,), jnp.float32),
        pltpu.VMEM((N_PAD,), jnp.float32),
        pltpu.SemaphoreType.DMA,
        pltpu.SemaphoreType.DMA((NBUF,)),
    ] + [pltpu.VMEM((CHUNK,), jnp.int32)] * NBUF
      + [pltpu.VMEM((CHUNK,), jnp.float32)] * NBUF
      + [pltpu.VMEM((CHUNK,), jnp.int32)] * NBUF,
)


def _sum_partials(part_ref):
    t = part_ref[:ROWS, :]
    for k in range(1, NW):
        t = t + part_ref[k * ROWS:(k + 1) * ROWS, :]
    return t


def _dense_body(part_ref, x_ref, a_ref, b_ref, c_ref, d_ref, e_ref,
                p_ref, r_ref):
    a1 = _sum_partials(part_ref)
    xv = x_ref[...]

    def body(k, carry):
        pacc, racc = carry
        h = jnp.maximum(a1 * a_ref[k] + xv * b_ref[k] + c_ref[k], 0.0)
        return (pacc + d_ref[k] * h, racc + e_ref[k] * h)

    z = jnp.zeros((ROWS, 128), jnp.float32)
    pv, rv = lax.fori_loop(0, HID, body, (z, z))
    p_ref[...] = pv
    r_ref[...] = rv


_dense = pl.pallas_call(
    _dense_body,
    out_shape=(jax.ShapeDtypeStruct((ROWS, 128), jnp.float32),
               jax.ShapeDtypeStruct((ROWS, 128), jnp.float32)),
    in_specs=[pl.BlockSpec(memory_space=pltpu.VMEM),
              pl.BlockSpec(memory_space=pltpu.VMEM)] +
             [pl.BlockSpec(memory_space=pltpu.SMEM)] * 5,
    out_specs=(pl.BlockSpec(memory_space=pltpu.VMEM),
               pl.BlockSpec(memory_space=pltpu.VMEM)),
)


def _final_body(part_ref, r_ref, b2_ref, o_ref):
    a2 = _sum_partials(part_ref)
    o_ref[...] = jax.nn.sigmoid(a2 + r_ref[...] + b2_ref[0])


_final = pl.pallas_call(
    _final_body,
    out_shape=jax.ShapeDtypeStruct((ROWS, 128), jnp.float32),
    in_specs=[pl.BlockSpec(memory_space=pltpu.VMEM),
              pl.BlockSpec(memory_space=pltpu.VMEM),
              pl.BlockSpec(memory_space=pltpu.SMEM)],
    out_specs=pl.BlockSpec(memory_space=pltpu.VMEM),
)


def kernel(x, edge_index, edge_weight, W1_rel, b1_rel, W1_root,
           W2_rel, b2_rel, W2_root):
    xf = x[:, 0]
    x_pad = jnp.pad(xf, (0, N_PAD - N_NODES_K))
    pad_e = E_PAD - N_EDGES_K
    src_p = jnp.pad(edge_index[0].astype(jnp.int32), (0, pad_e))
    dst_p = jnp.pad(edge_index[1].astype(jnp.int32), (0, pad_e))
    w_p = jnp.pad(edge_weight, (0, pad_e))

    part1 = _seg_kernel(x_pad, src_p, w_p, dst_p)
    p, r = _dense(part1.reshape(NW * ROWS, 128),
                  x_pad.reshape(ROWS, 128),
                  W1_rel[:, 0], W1_root[:, 0], b1_rel,
                  W2_rel[0], W2_root[0])
    part2 = _seg_kernel(p.reshape(-1), src_p, w_p, dst_p)
    out = _final(part2.reshape(NW * ROWS, 128), r, b2_rel)
    return out.reshape(N_PAD)[:N_NODES_K].reshape(N_NODES_K, 1)


# speedup vs baseline: 1.2304x; 1.2260x over previous
import functools

import jax
import jax.numpy as jnp
from jax import lax
from jax.experimental import pallas as pl
from jax.experimental.pallas import tpu as pltpu
from jax.experimental.pallas import tpu_sc as plsc

N_NODES_K = 50000
HID = 64
N_PAD = 50176
ROWS = N_PAD // 128
N_EDGES_K = 800000
NC, NS = 2, 16
NW = NC * NS
CHUNK = 1792
NCHUNK = 14
E_TILE = NCHUNK * CHUNK
E_PAD = NW * E_TILE
VSTEPS = CHUNK // 16
SROWS = CHUNK // 128
SLICE = N_PAD // NS


NBUF = 2
UNROLL = 4


def _seg_body(x_hbm, src_hbm, w_hbm, dst_hbm, out_hbm,
              x_v, acc_v, x_sem, in_sems, *bufs):
    src_v = bufs[0:NBUF]
    w_v = bufs[NBUF:2 * NBUF]
    dst_v = bufs[2 * NBUF:3 * NBUF]
    cid = lax.axis_index("c")
    sid = lax.axis_index("s")
    wid = cid * NS + sid

    x_cp = pltpu.async_copy(x_hbm, x_v, x_sem)

    def zbody(i, _):
        for u in range(UNROLL):
            acc_v[pl.ds(i * (16 * UNROLL) + u * 16, 16)] = (
                jnp.zeros((16,), jnp.float32))
        return 0
    lax.fori_loop(0, N_PAD // (16 * UNROLL), zbody, 0)
    x_cp.wait()

    def fire_inputs(c):
        b = c % NBUF
        base = wid * E_TILE + c * CHUNK
        return [
            pltpu.async_copy(src_hbm.at[pl.ds(base, CHUNK)],
                             src_v[b], in_sems.at[b]),
            pltpu.async_copy(w_hbm.at[pl.ds(base, CHUNK)],
                             w_v[b], in_sems.at[b]),
            pltpu.async_copy(dst_hbm.at[pl.ds(base, CHUNK)],
                             dst_v[b], in_sems.at[b]),
        ]

    in_cps = {0: fire_inputs(0), 1: fire_inputs(1)}
    for c in range(NCHUNK):
        b = c % NBUF
        for cp in in_cps.pop(c):
            cp.wait()

        def vbody(j, _):
            for u in range(UNROLL):
                o = j * (16 * UNROLL) + u * 16
                s = src_v[b][pl.ds(o, 16)]
                wv = w_v[b][pl.ds(o, 16)]
                d = dst_v[b][pl.ds(o, 16)]
                xv = plsc.load_gather(x_v, [s])
                plsc.addupdate_scatter(acc_v, [d], xv * wv)
            return 0
        lax.fori_loop(0, VSTEPS // UNROLL, vbody, 0)

        if c + 2 < NCHUNK:
            in_cps[c + 2] = fire_inputs(c + 2)

    pltpu.sync_copy(acc_v, out_hbm.at[pl.ds(wid * N_PAD, N_PAD)])


_seg_kernel = pl.kernel(
    _seg_body,
    out_type=jax.ShapeDtypeStruct((NW * N_PAD,), jnp.float32),
    mesh=plsc.VectorSubcoreMesh(core_axis_name="c", subcore_axis_name="s",
                                num_cores=NC, num_subcores=NS),
    compiler_params=pltpu.CompilerParams(needs_layout_passes=False),
    scratch_types=[
        pltpu.VMEM((N_PAD,), jnp.float32),
        pltpu.VMEM((N_PAD,), jnp.float32),
        pltpu.SemaphoreType.DMA,
        pltpu.SemaphoreType.DMA((NBUF,)),
    ] + [pltpu.VMEM((CHUNK,), jnp.int32)] * NBUF
      + [pltpu.VMEM((CHUNK,), jnp.float32)] * NBUF
      + [pltpu.VMEM((CHUNK,), jnp.int32)] * NBUF,
)


def _split_body(e_ref, s_ref, d_ref):
    s_ref[...] = e_ref[0, :]
    d_ref[...] = e_ref[1, :]


_split = pl.pallas_call(
    _split_body,
    in_specs=[pl.BlockSpec(memory_space=pltpu.VMEM)],
    out_specs=(pl.BlockSpec(memory_space=pltpu.VMEM),
               pl.BlockSpec(memory_space=pltpu.VMEM)),
    out_shape=(jax.ShapeDtypeStruct((N_EDGES_K,), jnp.int32),
               jax.ShapeDtypeStruct((N_EDGES_K,), jnp.int32)),
)


def _sum_partials(part_ref):
    t = part_ref[:ROWS, :]
    for k in range(1, NW):
        t = t + part_ref[k * ROWS:(k + 1) * ROWS, :]
    return t


def _dense_body(part_ref, x_ref, a_ref, b_ref, c_ref, d_ref, e_ref,
                p_ref, r_ref):
    a1 = _sum_partials(part_ref)
    xv = x_ref[...]

    def body(k, carry):
        pacc, racc = carry
        h = jnp.maximum(a1 * a_ref[k] + xv * b_ref[k] + c_ref[k], 0.0)
        return (pacc + d_ref[k] * h, racc + e_ref[k] * h)

    z = jnp.zeros((ROWS, 128), jnp.float32)
    pv, rv = lax.fori_loop(0, HID, body, (z, z))
    p_ref[...] = pv
    r_ref[...] = rv


_dense = pl.pallas_call(
    _dense_body,
    out_shape=(jax.ShapeDtypeStruct((ROWS, 128), jnp.float32),
               jax.ShapeDtypeStruct((ROWS, 128), jnp.float32)),
    in_specs=[pl.BlockSpec(memory_space=pltpu.VMEM),
              pl.BlockSpec(memory_space=pltpu.VMEM)] +
             [pl.BlockSpec(memory_space=pltpu.SMEM)] * 5,
    out_specs=(pl.BlockSpec(memory_space=pltpu.VMEM),
               pl.BlockSpec(memory_space=pltpu.VMEM)),
)


def _final_body(part_ref, r_ref, b2_ref, o_ref):
    a2 = _sum_partials(part_ref)
    o_ref[...] = jax.nn.sigmoid(a2 + r_ref[...] + b2_ref[0])


_final = pl.pallas_call(
    _final_body,
    out_shape=jax.ShapeDtypeStruct((ROWS, 128), jnp.float32),
    in_specs=[pl.BlockSpec(memory_space=pltpu.VMEM),
              pl.BlockSpec(memory_space=pltpu.VMEM),
              pl.BlockSpec(memory_space=pltpu.SMEM)],
    out_specs=pl.BlockSpec(memory_space=pltpu.VMEM),
)


def kernel(x, edge_index, edge_weight, W1_rel, b1_rel, W1_root,
           W2_rel, b2_rel, W2_root):
    xf = x[:, 0]
    x_pad = jnp.pad(xf, (0, N_PAD - N_NODES_K))
    pad_e = E_PAD - N_EDGES_K
    src, dst = _split(edge_index.astype(jnp.int32))
    src_p = jnp.pad(src, (0, pad_e))
    dst_p = jnp.pad(dst, (0, pad_e))
    w_p = jnp.pad(edge_weight, (0, pad_e))

    part1 = _seg_kernel(x_pad, src_p, w_p, dst_p)
    p, r = _dense(part1.reshape(NW * ROWS, 128),
                  x_pad.reshape(ROWS, 128),
                  W1_rel[:, 0], W1_root[:, 0], b1_rel,
                  W2_rel[0], W2_root[0])
    part2 = _seg_kernel(p.reshape(-1), src_p, w_p, dst_p)
    out = _final(part2.reshape(NW * ROWS, 128), r, b2_rel)
    return out.reshape(N_PAD)[:N_NODES_K].reshape(N_NODES_K, 1)
